# Initial kernel scaffold; baseline (speedup 1.0000x reference)
import jax
import jax.numpy as jnp
from jax.experimental import pallas as pl


def kernel(logits, targets):
    raise NotImplementedError("write your pallas kernel here")


# speedup vs baseline: 35.0179x; 35.0179x over previous
import functools

import jax
import jax.numpy as jnp
from jax import lax
from jax.experimental import pallas as pl
from jax.experimental.pallas import tpu as pltpu
from jax.experimental.pallas import tpu_sc as plsc

NBITS = 11
B = 1 << NBITS
TB = 2 * B
NLANE = 16
TABW = TB * NLANE
NIMG = 16
N = 512 * 512
HALF = N // 2
CHUNK = 16384
NCHUNK = HALF // CHUNK
INT_MIN = -(1 << 31)


def _prep_body(lg_ref, tg_ref, idx_ref, r_ref, p_ref):
    lg = lg_ref[0]
    tg = tg_ref[0]
    lf = tg.astype(jnp.float32)
    e = 1.0 - lg * (2.0 * lf - 1.0)
    r = jnp.maximum(e, 0.0)
    bits = lax.bitcast_convert_type(e, jnp.int32)
    neg = lax.shift_right_arithmetic(bits, 31)
    key = lax.bitwise_xor(bits, lax.bitwise_or(neg, jnp.int32(INT_MIN)))
    bucket = lax.bitwise_and(lax.shift_right_arithmetic(key, 32 - NBITS),
                             jnp.int32(B - 1))
    idx_ref[0] = bucket + tg * B
    r_ref[0] = r
    p_ref[...] = jnp.broadcast_to(jnp.sum(lf), (1, 8, 128))


def _prep(logits, targets):
    return pl.pallas_call(
        _prep_body,
        grid=(NIMG,),
        in_specs=[
            pl.BlockSpec((1, 512, 512), lambda i: (i, 0, 0)),
            pl.BlockSpec((1, 512, 512), lambda i: (i, 0, 0)),
        ],
        out_specs=[
            pl.BlockSpec((1, 512, 512), lambda i: (i, 0, 0)),
            pl.BlockSpec((1, 512, 512), lambda i: (i, 0, 0)),
            pl.BlockSpec((1, 8, 128), lambda i: (i, 0, 0)),
        ],
        out_shape=[
            jax.ShapeDtypeStruct((NIMG, 512, 512), jnp.int32),
            jax.ShapeDtypeStruct((NIMG, 512, 512), jnp.float32),
            jax.ShapeDtypeStruct((NIMG, 8, 128), jnp.float32),
        ],
    )(logits, targets)


_mesh = plsc.VectorSubcoreMesh(core_axis_name="c", subcore_axis_name="s")


@functools.partial(
    pl.kernel,
    out_type=jax.ShapeDtypeStruct((NIMG, 2, TB), jnp.float32),
    mesh=_mesh,
    compiler_params=pltpu.CompilerParams(needs_layout_passes=False),
    scratch_types=[
        pltpu.VMEM((CHUNK,), jnp.int32),
        pltpu.VMEM((CHUNK,), jnp.float32),
        pltpu.VMEM((TABW,), jnp.float32),
        pltpu.VMEM((TB,), jnp.float32),
    ],
)
def _hist(idx_hbm, r_hbm, out_hbm, idx_v, r_v, table, red_v):
    c = lax.axis_index("c")
    s = lax.axis_index("s")

    @plsc.parallel_loop(0, TABW // 16, unroll=8)
    def _zero(i):
        table[pl.ds(i * 16, 16)] = jnp.zeros((16,), jnp.float32)

    laneoff = lax.iota(jnp.int32, 16) * TB
    base = c * HALF

    def chunk_body(k, carry):
        pltpu.sync_copy(idx_hbm.at[s, pl.ds(base + k * CHUNK, CHUNK)], idx_v)
        pltpu.sync_copy(r_hbm.at[s, pl.ds(base + k * CHUNK, CHUNK)], r_v)

        @plsc.parallel_loop(0, CHUNK // 16, unroll=8)
        def _scatter(g):
            i16 = idx_v[pl.ds(g * 16, 16)]
            r16 = r_v[pl.ds(g * 16, 16)]
            plsc.addupdate_scatter(table, [i16 + laneoff], r16)

        return carry

    lax.fori_loop(0, NCHUNK, chunk_body, 0)

    @plsc.parallel_loop(0, TB // 16, unroll=4)
    def _reduce(v):
        acc = table[pl.ds(v * 16, 16)]
        for l in range(1, NLANE):
            acc = acc + table[pl.ds(l * TB + v * 16, 16)]
        red_v[pl.ds(v * 16, 16)] = acc
    pltpu.sync_copy(red_v, out_hbm.at[s, c])


def _decode_boundary(k):
    bits = jnp.where(k < 0,
                     lax.bitwise_xor(k, jnp.int32(INT_MIN)),
                     lax.bitwise_not(k))
    e = lax.bitcast_convert_type(bits, jnp.float32)
    big = jnp.float32(3.4e38)
    e = jnp.where(e != e, jnp.where(k < 0, big, -big), e)
    return jnp.clip(e, -big, big)


def _post_body(t_ref, p_ref, o_ref):
    T = t_ref[:, 0, :] + t_ref[:, 1, :]
    P = p_ref[:, 0, 0:1]
    Sn = T[:, :B]
    Sp = T[:, B:]

    b = lax.broadcasted_iota(jnp.int32, (1, B), 1)
    klo = lax.shift_left(b, jnp.int32(32 - NBITS))
    khi = lax.shift_left(b + 1, jnp.int32(32 - NBITS))
    elo = _decode_boundary(klo)
    ehi = jnp.where(b == B - 1, jnp.float32(3.4e38), _decode_boundary(khi))
    m = jnp.maximum(elo * 0.5 + ehi * 0.5, 0.0)
    inv_m = 1.0 / jnp.maximum(m, jnp.float32(1e-30))

    Cn = Sn * inv_m
    Cp = Sp * inv_m

    nrow = B // 128
    r128 = lax.broadcasted_iota(jnp.int32, (128, 128), 0)
    c128 = lax.broadcasted_iota(jnp.int32, (128, 128), 1)
    U128 = (r128 <= c128).astype(jnp.float32)
    rr = lax.broadcasted_iota(jnp.int32, (nrow, nrow), 0)
    cc = lax.broadcasted_iota(jnp.int32, (nrow, nrow), 1)
    SU = (rr < cc).astype(jnp.float32)

    def cumsum_incl(X):
        X3 = X.reshape(16 * nrow, 128)
        within = jax.lax.dot_general(
            X3, U128, (((1,), (0,)), ((), ())),
            preferred_element_type=jnp.float32).reshape(16, nrow, 128)
        rowsums = within[:, :, 127]
        offs = jax.lax.dot_general(
            rowsums, SU, (((1,), (0,)), ((), ())),
            preferred_element_type=jnp.float32)
        return (within + offs[:, :, None]).reshape(16, B)

    tot_n = jnp.sum(Cn, axis=1, keepdims=True)
    tot_p = jnp.sum(Cp, axis=1, keepdims=True)
    n_above = tot_n - cumsum_incl(Cn)
    p_above = tot_p - cumsum_incl(Cp)

    D = P + n_above
    t = Sp / D + Sn * (P - p_above - Cp) / (D * (D + Cn))
    o_ref[...] = (jnp.sum(t) / NIMG).reshape(1, 1)


def _post(T, P):
    return pl.pallas_call(
        _post_body,
        out_shape=jax.ShapeDtypeStruct((1, 1), jnp.float32),
    )(T, P)


def kernel(logits, targets):
    idx, r, P = _prep(logits, targets)
    T = _hist(idx.reshape(NIMG, N), r.reshape(NIMG, N))
    return _post(T, P).reshape(())


# speedup vs baseline: 42.0295x; 1.2002x over previous
import functools

import jax
import jax.numpy as jnp
from jax import lax
from jax.experimental import pallas as pl
from jax.experimental.pallas import tpu as pltpu
from jax.experimental.pallas import tpu_sc as plsc

NBITS = 11
B = 1 << NBITS
TB = 2 * B
NLANE = 16
TABW = TB * NLANE
NIMG = 16
N = 512 * 512
HALF = N // 2
CHUNK = 16384
NCHUNK = HALF // CHUNK
INT_MIN = -(1 << 31)

_mesh = plsc.VectorSubcoreMesh(core_axis_name="c", subcore_axis_name="s",
                               num_cores=2, num_subcores=16)


_HIST_KERNEL_ARGS = dict(
    out_type=[
        jax.ShapeDtypeStruct((NIMG, 2, TB), jnp.float32),
        jax.ShapeDtypeStruct((NIMG, 2, 16), jnp.float32),
    ],
    mesh=_mesh,
    compiler_params=pltpu.CompilerParams(needs_layout_passes=False),
    scratch_types=[
        pltpu.VMEM((CHUNK,), jnp.float32),
        pltpu.VMEM((CHUNK,), jnp.int32),
        pltpu.VMEM((TABW,), jnp.float32),
        pltpu.VMEM((TB,), jnp.float32),
        pltpu.VMEM((16,), jnp.float32),
    ],
)


def _hist_body(lg_hbm, tg_hbm, t_out, p_out, lg_v, tg_v, table, red_v, pcnt_v):
    c = lax.axis_index("c")
    s = lax.axis_index("s")

    @plsc.parallel_loop(0, TABW // 16, unroll=8)
    def _zero(i):
        table[pl.ds(i * 16, 16)] = jnp.zeros((16,), jnp.float32)

    laneoff = lax.iota(jnp.int32, 16) * TB
    base = c * HALF

    def chunk_body(k, pacc):
        pltpu.sync_copy(lg_hbm.at[s, pl.ds(base + k * CHUNK, CHUNK)], lg_v)
        pltpu.sync_copy(tg_hbm.at[s, pl.ds(base + k * CHUNK, CHUNK)], tg_v)

        @plsc.parallel_loop(0, CHUNK // 16, unroll=8, carry=pacc)
        def _scatter(g, acc):
            ti = tg_v[pl.ds(g * 16, 16)]
            lg16 = lg_v[pl.ds(g * 16, 16)]
            lf = ti.astype(jnp.float32)
            e = 1.0 - lg16 * (2.0 * lf - 1.0)
            r = jnp.maximum(e, 0.0)
            bits = plsc.bitcast(e, jnp.int32)
            neg = lax.shift_right_arithmetic(bits, 31)
            key = lax.bitwise_xor(bits, lax.bitwise_or(neg, jnp.int32(INT_MIN)))
            bucket = lax.bitwise_and(
                lax.shift_right_arithmetic(key, 32 - NBITS), jnp.int32(B - 1))
            idx = bucket + lax.shift_left(ti, NBITS) + laneoff
            plsc.addupdate_scatter(table, [idx], r)
            return acc + ti

        return _scatter

    pacc = lax.fori_loop(0, NCHUNK, chunk_body, jnp.zeros((16,), jnp.int32))

    @plsc.parallel_loop(0, TB // 16, unroll=4)
    def _reduce(v):
        acc = table[pl.ds(v * 16, 16)]
        for l in range(1, NLANE):
            acc = acc + table[pl.ds(l * TB + v * 16, 16)]
        red_v[pl.ds(v * 16, 16)] = acc

    pltpu.sync_copy(red_v, t_out.at[s, c])
    pcnt_v[...] = pacc.astype(jnp.float32)
    pltpu.sync_copy(pcnt_v, p_out.at[s, c])


_hist = pl.kernel(_hist_body, **_HIST_KERNEL_ARGS)


def _decode_boundary(k):
    bits = jnp.where(k < 0,
                     lax.bitwise_xor(k, jnp.int32(INT_MIN)),
                     lax.bitwise_not(k))
    e = lax.bitcast_convert_type(bits, jnp.float32)
    big = jnp.float32(3.4e38)
    e = jnp.where(e != e, jnp.where(k < 0, big, -big), e)
    return jnp.clip(e, -big, big)


def _post_body(t_ref, p_ref, o_ref):
    T = t_ref[:, 0, :] + t_ref[:, 1, :]
    P = jnp.sum(p_ref[...], axis=(1, 2)).reshape(16, 1)
    Sn = T[:, :B]
    Sp = T[:, B:]

    b = lax.broadcasted_iota(jnp.int32, (1, B), 1)
    klo = lax.shift_left(b, jnp.int32(32 - NBITS))
    khi = lax.shift_left(b + 1, jnp.int32(32 - NBITS))
    elo = _decode_boundary(klo)
    ehi = jnp.where(b == B - 1, jnp.float32(3.4e38), _decode_boundary(khi))
    m = jnp.maximum(elo * 0.5 + ehi * 0.5, 0.0)
    inv_m = 1.0 / jnp.maximum(m, jnp.float32(1e-30))

    Cn = Sn * inv_m
    Cp = Sp * inv_m

    nrow = B // 128
    r128 = lax.broadcasted_iota(jnp.int32, (128, 128), 0)
    c128 = lax.broadcasted_iota(jnp.int32, (128, 128), 1)
    U128 = (r128 <= c128).astype(jnp.float32)
    rr = lax.broadcasted_iota(jnp.int32, (nrow, nrow), 0)
    cc = lax.broadcasted_iota(jnp.int32, (nrow, nrow), 1)
    SU = (rr < cc).astype(jnp.float32)

    def cumsum_incl(X):
        X3 = X.reshape(16 * nrow, 128)
        within = jax.lax.dot_general(
            X3, U128, (((1,), (0,)), ((), ())),
            preferred_element_type=jnp.float32).reshape(16, nrow, 128)
        rowsums = within[:, :, 127]
        offs = jax.lax.dot_general(
            rowsums, SU, (((1,), (0,)), ((), ())),
            preferred_element_type=jnp.float32)
        return (within + offs[:, :, None]).reshape(16, B)

    tot_n = jnp.sum(Cn, axis=1, keepdims=True)
    tot_p = jnp.sum(Cp, axis=1, keepdims=True)
    n_above = tot_n - cumsum_incl(Cn)
    p_above = tot_p - cumsum_incl(Cp)

    D = P + n_above
    t = Sp / D + Sn * (P - p_above - Cp) / (D * (D + Cn))
    o_ref[...] = (jnp.sum(t) / NIMG).reshape(1, 1)


def _post(T, P):
    return pl.pallas_call(
        _post_body,
        out_shape=jax.ShapeDtypeStruct((1, 1), jnp.float32),
    )(T, P)


def kernel(logits, targets):
    T, Pp = _hist(logits.reshape(NIMG, N), targets.reshape(NIMG, N))
    return _post(T, Pp).reshape(())
